# baseline (speedup 1.0000x reference)
import functools

import jax
import jax.numpy as jnp
from jax import lax
from jax.experimental import pallas as pl
from jax.experimental.pallas import tpu as pltpu
from jax.experimental.pallas import tpu_sc as plsc


def _route_body(C, E, x_ref, wg_ref, bg_ref, slot_ref, scale_ref, xb_ref,
                counts_ref):
  i = pl.program_id(0)

  @pl.when(i == 0)
  def _():
    counts_ref[...] = jnp.zeros_like(counts_ref)

  logits_te = jnp.dot(x_ref[...], wg_ref[...],
                      preferred_element_type=jnp.float32) + bg_ref[...]
  logits = logits_te.T
  T = logits.shape[1]
  m = jnp.max(logits, axis=0, keepdims=True)
  gate = 1.0 / jnp.sum(jnp.exp(logits - m), axis=0, keepdims=True)
  e_iota = lax.broadcasted_iota(jnp.int32, logits.shape, 0)
  idx = jnp.min(jnp.where(logits == m, e_iota, E), axis=0,
                keepdims=True)
  onehot = (e_iota == idx).astype(jnp.float32)

  r = lax.broadcasted_iota(jnp.int32, (T, T), 0)
  c = lax.broadcasted_iota(jnp.int32, (T, T), 1)
  triu = (r < c).astype(jnp.float32)
  loc = jnp.dot(onehot, triu, preferred_element_type=jnp.float32)
  loc = loc + counts_ref[...]
  counts_ref[...] = counts_ref[...] + jnp.sum(onehot, axis=1, keepdims=True)
  loc_i = jnp.sum(loc * onehot, axis=0, keepdims=True).astype(jnp.int32)

  valid = loc_i < C
  slot = idx * C + loc_i
  slot_ref[...] = jnp.where(valid, slot, E * C)
  scale_ref[...] = gate
  xb_ref[...] = x_ref[...].astype(jnp.bfloat16)


def _route(xr, Wg, bg, C, E, block_t):
  S, M = xr.shape
  n = S // block_t
  return pl.pallas_call(
      functools.partial(_route_body, C, E),
      grid=(n,),
      in_specs=[
          pl.BlockSpec((block_t, M), lambda i: (i, 0)),
          pl.BlockSpec((M, E), lambda i: (0, 0)),
          pl.BlockSpec((1, E), lambda i: (0, 0)),
      ],
      out_specs=[
          pl.BlockSpec((1, block_t), lambda i: (0, i)),
          pl.BlockSpec((1, block_t), lambda i: (0, i)),
          pl.BlockSpec((block_t, M), lambda i: (i, 0)),
      ],
      out_shape=[
          jax.ShapeDtypeStruct((1, S), jnp.int32),
          jax.ShapeDtypeStruct((1, S), jnp.float32),
          jax.ShapeDtypeStruct((S, M), jnp.bfloat16),
      ],
      scratch_shapes=[pltpu.VMEM((E, 1), jnp.float32)],
  )(xr, Wg, bg.reshape(1, E))


def _ffn_body(ZE, x_ref, w1_ref, b1_ref, w2_ref, b2_ref, g_ref, out_ref):
  e = pl.program_id(0)

  @pl.when(e == ZE)
  def _():
    out_ref[...] = jnp.zeros_like(out_ref)

  @pl.when(e < ZE)
  def _():
    xb = x_ref[...]
    w1b = w1_ref[0].astype(jnp.bfloat16)
    h = jnp.dot(xb, w1b, preferred_element_type=jnp.float32)
    h = jnp.maximum(h + b1_ref[0], 0.0)
    w2b = w2_ref[0].astype(jnp.bfloat16)
    contrib = jnp.dot(h.astype(jnp.bfloat16), w2b,
                      preferred_element_type=jnp.float32)
    gcol = g_ref[...].reshape(1, x_ref.shape[0]).T
    out_ref[...] = (contrib + b2_ref[0]) * gcol


def _ffn_half(disp, W1, b1, W2, b2, gfs, C, e0, ne, eo_rows, eo_alias):
  E, M, F = W1.shape
  with_zero = eo_alias is not None
  grid = (ne + 1,) if with_zero else (ne,)
  ecl = lambda e: jnp.minimum(e, ne - 1)
  in_specs = [
      pl.BlockSpec((C, M), lambda e: (ecl(e), 0)),
      pl.BlockSpec((1, M, F), lambda e: (e0 + ecl(e), 0, 0)),
      pl.BlockSpec((1, 1, F), lambda e: (e0 + ecl(e), 0, 0)),
      pl.BlockSpec((1, F, M), lambda e: (e0 + ecl(e), 0, 0)),
      pl.BlockSpec((1, 1, M), lambda e: (e0 + ecl(e), 0, 0)),
      pl.BlockSpec((C,), lambda e: (ecl(e),)),
  ]
  args = [disp, W1, b1.reshape(E, 1, F), W2, b2.reshape(E, 1, M), gfs]
  kwargs = {}
  if with_zero:
    in_specs.append(pl.BlockSpec(memory_space=pltpu.MemorySpace.HBM))
    args.append(eo_alias)
    kwargs["input_output_aliases"] = {6: 0}
  body = functools.partial(_ffn_body, ne)
  if with_zero:
    def body(*refs):
      _ffn_body(ne, *refs[:6], refs[7])
  return pl.pallas_call(
      body,
      grid=grid,
      in_specs=in_specs,
      out_specs=pl.BlockSpec((C, M), lambda e: (e0 + e, 0)),
      out_shape=jax.ShapeDtypeStruct((eo_rows, M), jnp.float32),
      **kwargs,
  )(*args)


_NC = 2
_NS = 16
_NW = _NC * _NS


def _make_dispatch(S, M, n_table, slot_base, n_half, CH):
  mesh = plsc.VectorSubcoreMesh(core_axis_name="c", subcore_axis_name="s")
  TPT = S // _NS
  SPW = n_half // _NW
  TFSN = ((n_table + _NS * 16 - 1) // (_NS * 16)) * _NS * 16
  ZPT = TFSN // _NS

  @functools.partial(
      pl.kernel,
      mesh=mesh,
      out_type=(
          jax.ShapeDtypeStruct((n_half, M // 2), jnp.int32),
          jax.ShapeDtypeStruct((n_half,), jnp.float32),
      ),
      scratch_types=[
          pltpu.VMEM_SHARED((TFSN,), jnp.int32),
          pltpu.VMEM((ZPT,), jnp.int32),
          pltpu.VMEM((2, TPT // 2), jnp.int32),
          pltpu.VMEM((2, TPT // 2), jnp.int32),
          pltpu.VMEM((SPW,), jnp.int32),
          pltpu.VMEM((SPW,), jnp.int32),
          pltpu.VMEM((SPW,), jnp.float32),
          pltpu.VMEM((CH, M // 2), jnp.int32),
          pltpu.VMEM((CH, M // 2), jnp.int32),
          pltpu.SemaphoreType.DMA,
          pltpu.SemaphoreType.DMA,
          pltpu.SemaphoreType.DMA,
          pltpu.SemaphoreType.DMA,
          pltpu.SemaphoreType.DMA,
      ],
  )
  def dispatch(x_hbm, slot_hbm, scale_hbm, out_hbm, gfs_hbm, tfs_sh, zb_v,
               slots_v, ids_v, t_v, idxg_v, gsc_v, r0, r1, l0, l1, s0, s1,
               gs):
    cid = lax.axis_index("c")
    sid = lax.axis_index("s")
    wid = sid * _NC + cid
    for i in range(ZPT // 16):
      zb_v[pl.ds(i * 16, 16)] = jnp.zeros((16,), jnp.int32)
    pltpu.sync_copy(zb_v, tfs_sh.at[pl.ds(sid * ZPT, ZPT)])
    for k in range(2):
      pltpu.sync_copy(
          slot_hbm.at[pl.ds(sid * TPT + k * (TPT // 2), TPT // 2)],
          slots_v.at[k])
      for v in range(TPT // 32):
        ids_v[k, pl.ds(v * 16, 16)] = (
            lax.iota(jnp.int32, 16) + (sid * TPT + k * (TPT // 2) + v * 16
                                       + 1))
    plsc.subcore_barrier()
    for k in range(2):
      pltpu.sync_copy(ids_v.at[k], tfs_sh.at[slots_v.at[k]], add=True)
    plsc.subcore_barrier()
    sbase = wid * SPW
    pltpu.sync_copy(tfs_sh.at[pl.ds(slot_base + sbase, SPW)], t_v)
    for v in range(SPW // 16):
      tv = t_v[pl.ds(v * 16, 16)]
      idxg_v[pl.ds(v * 16, 16)] = jnp.maximum(tv, 1) - 1
    gcopy = pltpu.async_copy(scale_hbm.at[idxg_v], gsc_v, gs)
    bufs, lsem, ssem = (r0, r1), (l0, l1), (s0, s1)
    stores = [None, None]
    for j in range(SPW // CH):
      b = j % 2
      if stores[b] is not None:
        stores[b].wait()
      pltpu.async_copy(x_hbm.at[idxg_v.at[pl.ds(j * CH, CH)]], bufs[b],
                       lsem[b]).wait()
      stores[b] = pltpu.async_copy(bufs[b],
                                   out_hbm.at[pl.ds(sbase + j * CH, CH)],
                                   ssem[b])
    gcopy.wait()
    pltpu.sync_copy(gsc_v, gfs_hbm.at[pl.ds(sbase, SPW)])
    for s in stores:
      s.wait()

  return dispatch


def _make_decode(S, M, K, CH):
  mesh = plsc.VectorSubcoreMesh(core_axis_name="c", subcore_axis_name="s")

  @functools.partial(
      pl.kernel,
      mesh=mesh,
      out_type=jax.ShapeDtypeStruct((S, M), jnp.float32),
      scratch_types=[
          pltpu.VMEM((K, CH), jnp.int32),
          pltpu.VMEM((CH, M), jnp.float32),
          pltpu.VMEM((CH, M), jnp.float32),
          pltpu.SemaphoreType.DMA,
          pltpu.SemaphoreType.DMA,
          pltpu.SemaphoreType.DMA,
          pltpu.SemaphoreType.DMA,
      ],
  )
  def decode(eo_hbm, slot_hbm, out_hbm, idx_v, r0, r1, g0, g1, s0, s1):
    wid = lax.axis_index("s") * _NC + lax.axis_index("c")
    tbase = wid * (K * CH)
    for j in range(K):
      pltpu.sync_copy(slot_hbm.at[pl.ds(tbase + j * CH, CH)], idx_v.at[j])
    bufs, gsem, ssem = (r0, r1), (g0, g1), (s0, s1)
    stores = [None, None]
    for j in range(K):
      b = j % 2
      if stores[b] is not None:
        stores[b].wait()
      base = wid * (K * CH) + j * CH
      pltpu.async_copy(eo_hbm.at[idx_v.at[j]], bufs[b], gsem[b]).wait()
      stores[b] = pltpu.async_copy(bufs[b], out_hbm.at[pl.ds(base, CH)],
                                   ssem[b])
    for s in stores:
      s.wait()

  return decode


def kernel(x, Wg, bg, W1, b1, W2, b2):
  orig_shape = x.shape
  M = x.shape[-1]
  xr = x.reshape(-1, M)
  S = xr.shape[0]
  E = Wg.shape[1]
  C = (S + E - 1) // E
  n_table = E * C + 8
  eo_rows = E * C + 8

  K, CH = 4, 32
  assert S == _NW * K * CH

  slot, scale, xb = _route(xr, Wg, bg, C, E, block_t=1024)
  slot1 = slot.reshape(S)
  scale1 = scale.reshape(S)

  half = S // 2
  xb32 = lax.bitcast_convert_type(xb.reshape(S, M // 2, 2), jnp.int32)
  dispA, gfsA = _make_dispatch(S, M, n_table, 0, half, CH)(xb32, slot1,
                                                           scale1)
  dispB, gfsB = _make_dispatch(S, M, n_table, half, half, CH)(xb32, slot1,
                                                              scale1)
  tobf = lambda d: lax.bitcast_convert_type(d, jnp.bfloat16).reshape(half, M)
  eoA = _ffn_half(tobf(dispA), W1, b1, W2, b2, gfsA, C, 0, E // 2, eo_rows,
                  None)
  eo = _ffn_half(tobf(dispB), W1, b1, W2, b2, gfsB, C, E // 2, E // 2,
                 eo_rows, eoA)
  rout = _make_decode(S, M, K, CH)(eo, slot1)
  return rout.reshape(orig_shape)


# speedup vs baseline: 2.2973x; 2.2973x over previous
import functools

import jax
import jax.numpy as jnp
from jax import lax
from jax.experimental import pallas as pl
from jax.experimental.pallas import tpu as pltpu
from jax.experimental.pallas import tpu_sc as plsc


def _route_body(C, E, x_ref, wg_ref, bg_ref, slot_ref, scale_ref, counts_ref):
  i = pl.program_id(0)

  @pl.when(i == 0)
  def _():
    counts_ref[...] = jnp.zeros_like(counts_ref)

  logits_te = jnp.dot(x_ref[...], wg_ref[...],
                      preferred_element_type=jnp.float32) + bg_ref[...]
  logits = logits_te.T
  T = logits.shape[1]
  m = jnp.max(logits, axis=0, keepdims=True)
  gate = 1.0 / jnp.sum(jnp.exp(logits - m), axis=0, keepdims=True)
  e_iota = lax.broadcasted_iota(jnp.int32, logits.shape, 0)
  idx = jnp.min(jnp.where(logits == m, e_iota, E), axis=0,
                keepdims=True)
  onehot = (e_iota == idx).astype(jnp.float32)

  r = lax.broadcasted_iota(jnp.int32, (T, T), 0)
  c = lax.broadcasted_iota(jnp.int32, (T, T), 1)
  triu = (r < c).astype(jnp.float32)
  loc = jnp.dot(onehot, triu, preferred_element_type=jnp.float32)
  loc = loc + counts_ref[...]
  counts_ref[...] = counts_ref[...] + jnp.sum(onehot, axis=1, keepdims=True)
  loc_i = jnp.sum(loc * onehot, axis=0, keepdims=True).astype(jnp.int32)

  valid = loc_i < C
  slot = idx * C + loc_i
  slot_ref[...] = jnp.where(valid, slot, E * C)
  scale_ref[...] = gate


def _route(xr, Wg, bg, C, E, block_t):
  S, M = xr.shape
  n = S // block_t
  return pl.pallas_call(
      functools.partial(_route_body, C, E),
      grid=(n,),
      in_specs=[
          pl.BlockSpec((block_t, M), lambda i: (i, 0)),
          pl.BlockSpec((M, E), lambda i: (0, 0)),
          pl.BlockSpec((1, E), lambda i: (0, 0)),
      ],
      out_specs=[
          pl.BlockSpec((1, block_t), lambda i: (0, i)),
          pl.BlockSpec((1, block_t), lambda i: (0, i)),
      ],
      out_shape=[
          jax.ShapeDtypeStruct((1, S), jnp.int32),
          jax.ShapeDtypeStruct((1, S), jnp.float32),
      ],
      scratch_shapes=[pltpu.VMEM((E, 1), jnp.float32)],
  )(xr, Wg, bg.reshape(1, E))


def _ffn_body(ZE, x_ref, w1_ref, b1_ref, w2_ref, b2_ref, g_ref, out_ref):
  e = pl.program_id(0)

  @pl.when(e == ZE)
  def _():
    out_ref[...] = jnp.zeros_like(out_ref)

  @pl.when(e < ZE)
  def _():
    xb = x_ref[...].astype(jnp.bfloat16)
    w1b = w1_ref[0].astype(jnp.bfloat16)
    h = jnp.dot(xb, w1b, preferred_element_type=jnp.float32)
    h = jnp.maximum(h + b1_ref[0], 0.0)
    w2b = w2_ref[0].astype(jnp.bfloat16)
    contrib = jnp.dot(h.astype(jnp.bfloat16), w2b,
                      preferred_element_type=jnp.float32)
    gcol = g_ref[...].reshape(1, x_ref.shape[0]).T
    out_ref[...] = (contrib + b2_ref[0]) * gcol


def _ffn_half(disp, W1, b1, W2, b2, gfs, C, e0, ne, eo_rows, eo_alias):
  E, M, F = W1.shape
  with_zero = (e0 + ne == E)
  grid = (ne + 1,) if with_zero else (ne,)
  ecl = lambda e: jnp.minimum(e, ne - 1)
  in_specs = [
      pl.BlockSpec((C, M), lambda e: (ecl(e), 0)),
      pl.BlockSpec((1, M, F), lambda e: (e0 + ecl(e), 0, 0)),
      pl.BlockSpec((1, 1, F), lambda e: (e0 + ecl(e), 0, 0)),
      pl.BlockSpec((1, F, M), lambda e: (e0 + ecl(e), 0, 0)),
      pl.BlockSpec((1, 1, M), lambda e: (e0 + ecl(e), 0, 0)),
      pl.BlockSpec((C,), lambda e: (ecl(e),)),
  ]
  args = [disp, W1, b1.reshape(E, 1, F), W2, b2.reshape(E, 1, M), gfs]
  kwargs = {}
  body = functools.partial(_ffn_body, ne)
  if eo_alias is not None:
    in_specs.append(pl.BlockSpec(memory_space=pltpu.MemorySpace.HBM))
    args.append(eo_alias)
    kwargs["input_output_aliases"] = {6: 0}
    def body(*refs):
      _ffn_body(ne, *refs[:6], refs[7])
  return pl.pallas_call(
      body,
      grid=grid,
      in_specs=in_specs,
      out_specs=pl.BlockSpec((C, M), lambda e: (e0 + e, 0)),
      out_shape=jax.ShapeDtypeStruct((eo_rows, M), jnp.float32),
      **kwargs,
  )(*args)


_NC = 2
_NS = 16
_NW = _NC * _NS


def _make_dispatch(S, M, n_table, slot_base, n_half, CH):
  mesh = plsc.VectorSubcoreMesh(core_axis_name="c", subcore_axis_name="s")
  TPT = S // _NS
  SPW = n_half // _NW
  TFSN = ((n_table + _NS * 16 - 1) // (_NS * 16)) * _NS * 16
  ZPT = TFSN // _NS

  @functools.partial(
      pl.kernel,
      mesh=mesh,
      out_type=(
          jax.ShapeDtypeStruct((n_half, M), jnp.float32),
          jax.ShapeDtypeStruct((n_half,), jnp.float32),
      ),
      scratch_types=[
          pltpu.VMEM_SHARED((TFSN,), jnp.int32),
          pltpu.VMEM((ZPT,), jnp.int32),
          pltpu.VMEM((2, TPT // 2), jnp.int32),
          pltpu.VMEM((2, TPT // 2), jnp.int32),
          pltpu.VMEM((SPW,), jnp.int32),
          pltpu.VMEM((SPW,), jnp.int32),
          pltpu.VMEM((SPW,), jnp.float32),
          pltpu.VMEM((CH, M), jnp.float32),
          pltpu.VMEM((CH, M), jnp.float32),
          pltpu.SemaphoreType.DMA,
          pltpu.SemaphoreType.DMA,
          pltpu.SemaphoreType.DMA,
          pltpu.SemaphoreType.DMA,
          pltpu.SemaphoreType.DMA,
      ],
  )
  def dispatch(x_hbm, slot_hbm, scale_hbm, out_hbm, gfs_hbm, tfs_sh, zb_v,
               slots_v, ids_v, t_v, idxg_v, gsc_v, r0, r1, l0, l1, s0, s1,
               gs):
    cid = lax.axis_index("c")
    sid = lax.axis_index("s")
    wid = sid * _NC + cid
    for i in range(ZPT // 16):
      zb_v[pl.ds(i * 16, 16)] = jnp.zeros((16,), jnp.int32)
    pltpu.sync_copy(zb_v, tfs_sh.at[pl.ds(sid * ZPT, ZPT)])
    for k in range(2):
      pltpu.sync_copy(
          slot_hbm.at[pl.ds(sid * TPT + k * (TPT // 2), TPT // 2)],
          slots_v.at[k])
      for v in range(TPT // 32):
        ids_v[k, pl.ds(v * 16, 16)] = (
            lax.iota(jnp.int32, 16) + (sid * TPT + k * (TPT // 2) + v * 16
                                       + 1))
    plsc.subcore_barrier()
    for k in range(2):
      pltpu.sync_copy(ids_v.at[k], tfs_sh.at[slots_v.at[k]], add=True)
    plsc.subcore_barrier()
    sbase = wid * SPW
    pltpu.sync_copy(tfs_sh.at[pl.ds(slot_base + sbase, SPW)], t_v)
    for v in range(SPW // 16):
      tv = t_v[pl.ds(v * 16, 16)]
      idxg_v[pl.ds(v * 16, 16)] = jnp.maximum(tv, 1) - 1
    gcopy = pltpu.async_copy(scale_hbm.at[idxg_v], gsc_v, gs)
    bufs, lsem, ssem = (r0, r1), (l0, l1), (s0, s1)
    stores = [None, None]
    for j in range(SPW // CH):
      b = j % 2
      if stores[b] is not None:
        stores[b].wait()
      pltpu.async_copy(x_hbm.at[idxg_v.at[pl.ds(j * CH, CH)]], bufs[b],
                       lsem[b]).wait()
      stores[b] = pltpu.async_copy(bufs[b],
                                   out_hbm.at[pl.ds(sbase + j * CH, CH)],
                                   ssem[b])
    gcopy.wait()
    pltpu.sync_copy(gsc_v, gfs_hbm.at[pl.ds(sbase, SPW)])
    for s in stores:
      s.wait()

  return dispatch


def _make_decode(S, M, K, CH):
  mesh = plsc.VectorSubcoreMesh(core_axis_name="c", subcore_axis_name="s")

  @functools.partial(
      pl.kernel,
      mesh=mesh,
      out_type=jax.ShapeDtypeStruct((S, M), jnp.float32),
      scratch_types=[
          pltpu.VMEM((K, CH), jnp.int32),
          pltpu.VMEM((CH, M), jnp.float32),
          pltpu.VMEM((CH, M), jnp.float32),
          pltpu.SemaphoreType.DMA,
          pltpu.SemaphoreType.DMA,
          pltpu.SemaphoreType.DMA,
          pltpu.SemaphoreType.DMA,
      ],
  )
  def decode(eo_hbm, slot_hbm, out_hbm, idx_v, r0, r1, g0, g1, s0, s1):
    wid = lax.axis_index("s") * _NC + lax.axis_index("c")
    tbase = wid * (K * CH)
    for j in range(K):
      pltpu.sync_copy(slot_hbm.at[pl.ds(tbase + j * CH, CH)], idx_v.at[j])
    bufs, gsem, ssem = (r0, r1), (g0, g1), (s0, s1)
    stores = [None, None]
    for j in range(K):
      b = j % 2
      if stores[b] is not None:
        stores[b].wait()
      base = wid * (K * CH) + j * CH
      pltpu.async_copy(eo_hbm.at[idx_v.at[j]], bufs[b], gsem[b]).wait()
      stores[b] = pltpu.async_copy(bufs[b], out_hbm.at[pl.ds(base, CH)],
                                   ssem[b])
    for s in stores:
      s.wait()

  return decode


def kernel(x, Wg, bg, W1, b1, W2, b2):
  orig_shape = x.shape
  M = x.shape[-1]
  xr = x.reshape(-1, M)
  S = xr.shape[0]
  E = Wg.shape[1]
  C = (S + E - 1) // E
  n_table = E * C + 8
  eo_rows = E * C + 8

  K, CH = 4, 32
  assert S == _NW * K * CH

  slot, scale = _route(xr, Wg, bg, C, E, block_t=1024)
  slot1 = slot.reshape(S)
  scale1 = scale.reshape(S)

  dispF, gfsF = _make_dispatch(S, M, n_table, 0, S, CH)(xr, slot1, scale1)
  eo = _ffn_half(dispF, W1, b1, W2, b2, gfsF, C, 0, E, eo_rows, None)
  rout = _make_decode(S, M, K, CH)(eo, slot1)
  return rout.reshape(orig_shape)


# speedup vs baseline: 2.3135x; 1.0070x over previous
import functools

import jax
import jax.numpy as jnp
from jax import lax
from jax.experimental import pallas as pl
from jax.experimental.pallas import tpu as pltpu
from jax.experimental.pallas import tpu_sc as plsc


def _route_body(C, E, x_ref, wg_ref, bg_ref, slot_ref, scale_ref, counts_ref):
  i = pl.program_id(0)

  @pl.when(i == 0)
  def _():
    counts_ref[...] = jnp.zeros_like(counts_ref)

  logits_te = jnp.dot(x_ref[...], wg_ref[...],
                      preferred_element_type=jnp.float32) + bg_ref[...]
  logits = logits_te.T
  T = logits.shape[1]
  m = jnp.max(logits, axis=0, keepdims=True)
  gate = 1.0 / jnp.sum(jnp.exp(logits - m), axis=0, keepdims=True)
  e_iota = lax.broadcasted_iota(jnp.int32, logits.shape, 0)
  idx = jnp.min(jnp.where(logits == m, e_iota, E), axis=0,
                keepdims=True)
  onehot = (e_iota == idx).astype(jnp.float32)

  r = lax.broadcasted_iota(jnp.int32, (T, T), 0)
  c = lax.broadcasted_iota(jnp.int32, (T, T), 1)
  triu = (r < c).astype(jnp.float32)
  loc = jnp.dot(onehot, triu, preferred_element_type=jnp.float32)
  loc = loc + counts_ref[...]
  counts_ref[...] = counts_ref[...] + jnp.sum(onehot, axis=1, keepdims=True)
  loc_i = jnp.sum(loc * onehot, axis=0, keepdims=True).astype(jnp.int32)

  valid = loc_i < C
  slot = idx * C + loc_i
  slot_ref[...] = jnp.where(valid, slot, E * C)
  scale_ref[...] = gate


def _route(xr, Wg, bg, C, E, block_t):
  S, M = xr.shape
  n = S // block_t
  return pl.pallas_call(
      functools.partial(_route_body, C, E),
      grid=(n,),
      in_specs=[
          pl.BlockSpec((block_t, M), lambda i: (i, 0)),
          pl.BlockSpec((M, E), lambda i: (0, 0)),
          pl.BlockSpec((1, E), lambda i: (0, 0)),
      ],
      out_specs=[
          pl.BlockSpec((1, block_t), lambda i: (0, i)),
          pl.BlockSpec((1, block_t), lambda i: (0, i)),
      ],
      out_shape=[
          jax.ShapeDtypeStruct((1, S), jnp.int32),
          jax.ShapeDtypeStruct((1, S), jnp.float32),
      ],
      scratch_shapes=[pltpu.VMEM((E, 1), jnp.float32)],
  )(xr, Wg, bg.reshape(1, E))


def _ffn_body(ZE, x_ref, w1_ref, b1_ref, w2_ref, b2_ref, g_ref, out_ref):
  e = pl.program_id(0)

  @pl.when(e == ZE)
  def _():
    out_ref[...] = jnp.zeros_like(out_ref)

  @pl.when(e < ZE)
  def _():
    xb = x_ref[...].astype(jnp.bfloat16)
    w1b = w1_ref[0].astype(jnp.bfloat16)
    h = jnp.dot(xb, w1b, preferred_element_type=jnp.float32)
    h = jnp.maximum(h + b1_ref[0], 0.0)
    w2b = w2_ref[0].astype(jnp.bfloat16)
    contrib = jnp.dot(h.astype(jnp.bfloat16), w2b,
                      preferred_element_type=jnp.float32)
    gcol = g_ref[...].reshape(1, x_ref.shape[0]).T
    out_ref[...] = (contrib + b2_ref[0]) * gcol


def _ffn_half(disp, W1, b1, W2, b2, gfs, C, e0, ne, eo_rows, eo_alias):
  E, M, F = W1.shape
  with_zero = (e0 + ne == E)
  grid = (ne + 1,) if with_zero else (ne,)
  ecl = lambda e: jnp.minimum(e, ne - 1)
  in_specs = [
      pl.BlockSpec((C, M), lambda e: (ecl(e), 0)),
      pl.BlockSpec((1, M, F), lambda e: (e0 + ecl(e), 0, 0)),
      pl.BlockSpec((1, 1, F), lambda e: (e0 + ecl(e), 0, 0)),
      pl.BlockSpec((1, F, M), lambda e: (e0 + ecl(e), 0, 0)),
      pl.BlockSpec((1, 1, M), lambda e: (e0 + ecl(e), 0, 0)),
      pl.BlockSpec((C,), lambda e: (ecl(e),)),
  ]
  args = [disp, W1, b1.reshape(E, 1, F), W2, b2.reshape(E, 1, M), gfs]
  kwargs = {}
  body = functools.partial(_ffn_body, ne)
  if eo_alias is not None:
    in_specs.append(pl.BlockSpec(memory_space=pltpu.MemorySpace.HBM))
    args.append(eo_alias)
    kwargs["input_output_aliases"] = {6: 0}
    def body(*refs):
      _ffn_body(ne, *refs[:6], refs[7])
  return pl.pallas_call(
      body,
      grid=grid,
      in_specs=in_specs,
      out_specs=pl.BlockSpec((C, M), lambda e: (e0 + e, 0)),
      out_shape=jax.ShapeDtypeStruct((eo_rows, M), jnp.float32),
      **kwargs,
  )(*args)


_NC = 2
_NS = 16
_NW = _NC * _NS


def _make_dispatch(S, M, n_table, slot_base, n_half, CH):
  mesh = plsc.VectorSubcoreMesh(core_axis_name="c", subcore_axis_name="s")
  TPT = S // _NS
  SPW = n_half // _NW
  TFSN = ((n_table + _NS * 16 - 1) // (_NS * 16)) * _NS * 16
  ZPT = TFSN // _NS

  @functools.partial(
      pl.kernel,
      mesh=mesh,
      out_type=(
          jax.ShapeDtypeStruct((n_half, M), jnp.float32),
          jax.ShapeDtypeStruct((n_half,), jnp.float32),
      ),
      scratch_types=[
          pltpu.VMEM_SHARED((TFSN,), jnp.int32),
          pltpu.VMEM((ZPT,), jnp.int32),
          pltpu.VMEM((2, TPT // 2), jnp.int32),
          pltpu.VMEM((2, TPT // 2), jnp.int32),
          pltpu.VMEM((SPW,), jnp.int32),
          pltpu.VMEM((SPW,), jnp.int32),
          pltpu.VMEM((SPW,), jnp.float32),
          pltpu.VMEM((CH, M), jnp.float32),
          pltpu.VMEM((CH, M), jnp.float32),
          pltpu.SemaphoreType.DMA,
          pltpu.SemaphoreType.DMA,
          pltpu.SemaphoreType.DMA,
          pltpu.SemaphoreType.DMA,
          pltpu.SemaphoreType.DMA,
      ],
  )
  def dispatch(x_hbm, slot_hbm, scale_hbm, out_hbm, gfs_hbm, tfs_sh, zb_v,
               slots_v, ids_v, t_v, idxg_v, gsc_v, r0, r1, l0, l1, s0, s1,
               gs):
    cid = lax.axis_index("c")
    sid = lax.axis_index("s")
    wid = sid * _NC + cid
    for i in range(ZPT // 16):
      zb_v[pl.ds(i * 16, 16)] = jnp.zeros((16,), jnp.int32)
    pltpu.sync_copy(zb_v, tfs_sh.at[pl.ds(sid * ZPT, ZPT)])
    for k in range(2):
      pltpu.sync_copy(
          slot_hbm.at[pl.ds(sid * TPT + k * (TPT // 2), TPT // 2)],
          slots_v.at[k])
      for v in range(TPT // 32):
        ids_v[k, pl.ds(v * 16, 16)] = (
            lax.iota(jnp.int32, 16) + (sid * TPT + k * (TPT // 2) + v * 16
                                       + 1))
    plsc.subcore_barrier()
    for k in range(2):
      pltpu.sync_copy(ids_v.at[k], tfs_sh.at[slots_v.at[k]], add=True)
    plsc.subcore_barrier()
    sbase = wid * SPW
    pltpu.sync_copy(tfs_sh.at[pl.ds(slot_base + sbase, SPW)], t_v)
    for v in range(SPW // 16):
      tv = t_v[pl.ds(v * 16, 16)]
      idxg_v[pl.ds(v * 16, 16)] = jnp.maximum(tv, 1) - 1
    gcopy = pltpu.async_copy(scale_hbm.at[idxg_v], gsc_v, gs)
    bufs, lsem, ssem = (r0, r1), (l0, l1), (s0, s1)
    stores = [None, None]
    for j in range(SPW // CH):
      b = j % 2
      if stores[b] is not None:
        stores[b].wait()
      pltpu.async_copy(x_hbm.at[idxg_v.at[pl.ds(j * CH, CH)]], bufs[b],
                       lsem[b]).wait()
      stores[b] = pltpu.async_copy(bufs[b],
                                   out_hbm.at[pl.ds(sbase + j * CH, CH)],
                                   ssem[b])
    gcopy.wait()
    pltpu.sync_copy(gsc_v, gfs_hbm.at[pl.ds(sbase, SPW)])
    for s in stores:
      s.wait()

  return dispatch


def _make_decode(S, M, K, CH):
  mesh = plsc.VectorSubcoreMesh(core_axis_name="c", subcore_axis_name="s")

  @functools.partial(
      pl.kernel,
      mesh=mesh,
      out_type=jax.ShapeDtypeStruct((S, M), jnp.float32),
      scratch_types=[
          pltpu.VMEM((K * CH,), jnp.int32),
          pltpu.VMEM((CH, M), jnp.float32),
          pltpu.VMEM((CH, M), jnp.float32),
          pltpu.SemaphoreType.DMA,
          pltpu.SemaphoreType.DMA,
          pltpu.SemaphoreType.DMA,
          pltpu.SemaphoreType.DMA,
      ],
  )
  def decode(eo_hbm, slot_hbm, out_hbm, idx_v, r0, r1, g0, g1, s0, s1):
    wid = lax.axis_index("s") * _NC + lax.axis_index("c")
    tbase = wid * (K * CH)
    pltpu.sync_copy(slot_hbm.at[pl.ds(tbase, K * CH)], idx_v)
    bufs, gsem, ssem = (r0, r1), (g0, g1), (s0, s1)
    stores = [None, None]
    for j in range(K):
      b = j % 2
      if stores[b] is not None:
        stores[b].wait()
      base = wid * (K * CH) + j * CH
      pltpu.async_copy(eo_hbm.at[idx_v.at[pl.ds(j * CH, CH)]], bufs[b],
                       gsem[b]).wait()
      stores[b] = pltpu.async_copy(bufs[b], out_hbm.at[pl.ds(base, CH)],
                                   ssem[b])
    for s in stores:
      s.wait()

  return decode


def kernel(x, Wg, bg, W1, b1, W2, b2):
  orig_shape = x.shape
  M = x.shape[-1]
  xr = x.reshape(-1, M)
  S = xr.shape[0]
  E = Wg.shape[1]
  C = (S + E - 1) // E
  n_table = E * C + 8
  eo_rows = E * C + 8

  K, CH = 4, 32
  assert S == _NW * K * CH

  slot, scale = _route(xr, Wg, bg, C, E, block_t=1024)
  slot1 = slot.reshape(S)
  scale1 = scale.reshape(S)

  dispF, gfsF = _make_dispatch(S, M, n_table, 0, S, CH)(xr, slot1, scale1)
  eo = _ffn_half(dispF, W1, b1, W2, b2, gfsF, C, 0, E, eo_rows, None)
  rout = _make_decode(S, M, K, CH)(eo, slot1)
  return rout.reshape(orig_shape)


# speedup vs baseline: 2.3173x; 1.0016x over previous
import functools

import jax
import jax.numpy as jnp
from jax import lax
from jax.experimental import pallas as pl
from jax.experimental.pallas import tpu as pltpu
from jax.experimental.pallas import tpu_sc as plsc


def _route_body(C, E, x_ref, wg_ref, bg_ref, slot_ref, scale_ref, counts_ref):
  i = pl.program_id(0)

  @pl.when(i == 0)
  def _():
    counts_ref[...] = jnp.zeros_like(counts_ref)

  logits_te = jnp.dot(x_ref[...], wg_ref[...],
                      preferred_element_type=jnp.float32) + bg_ref[...]
  logits = logits_te.T
  T = logits.shape[1]
  m = jnp.max(logits, axis=0, keepdims=True)
  gate = 1.0 / jnp.sum(jnp.exp(logits - m), axis=0, keepdims=True)
  e_iota = lax.broadcasted_iota(jnp.int32, logits.shape, 0)
  idx = jnp.min(jnp.where(logits == m, e_iota, E), axis=0,
                keepdims=True)
  onehot = (e_iota == idx).astype(jnp.float32)

  r = lax.broadcasted_iota(jnp.int32, (T, T), 0)
  c = lax.broadcasted_iota(jnp.int32, (T, T), 1)
  triu = (r < c).astype(jnp.float32)
  loc = jnp.dot(onehot, triu, preferred_element_type=jnp.float32)
  loc = loc + counts_ref[...]
  counts_ref[...] = counts_ref[...] + jnp.sum(onehot, axis=1, keepdims=True)
  loc_i = jnp.sum(loc * onehot, axis=0, keepdims=True).astype(jnp.int32)

  valid = loc_i < C
  slot = idx * C + loc_i
  slot_ref[...] = jnp.where(valid, slot, E * C)
  scale_ref[...] = gate


def _route(xr, Wg, bg, C, E, block_t):
  S, M = xr.shape
  n = S // block_t
  return pl.pallas_call(
      functools.partial(_route_body, C, E),
      grid=(n,),
      in_specs=[
          pl.BlockSpec((block_t, M), lambda i: (i, 0)),
          pl.BlockSpec((M, E), lambda i: (0, 0)),
          pl.BlockSpec((1, E), lambda i: (0, 0)),
      ],
      out_specs=[
          pl.BlockSpec((1, block_t), lambda i: (0, i)),
          pl.BlockSpec((1, block_t), lambda i: (0, i)),
      ],
      out_shape=[
          jax.ShapeDtypeStruct((1, S), jnp.int32),
          jax.ShapeDtypeStruct((1, S), jnp.float32),
      ],
      scratch_shapes=[pltpu.VMEM((E, 1), jnp.float32)],
  )(xr, Wg, bg.reshape(1, E))


def _ffn_body(ZE, x_ref, w1_ref, b1_ref, w2_ref, b2_ref, g_ref, out_ref):
  e = pl.program_id(0)

  @pl.when(e == ZE)
  def _():
    out_ref[...] = jnp.zeros_like(out_ref)

  @pl.when(e < ZE)
  def _():
    xb = x_ref[...].astype(jnp.bfloat16)
    w1b = w1_ref[0].astype(jnp.bfloat16)
    h = jnp.dot(xb, w1b, preferred_element_type=jnp.float32)
    h = jnp.maximum(h + b1_ref[0], 0.0)
    w2b = w2_ref[0].astype(jnp.bfloat16)
    contrib = jnp.dot(h.astype(jnp.bfloat16), w2b,
                      preferred_element_type=jnp.float32)
    gcol = g_ref[...].reshape(1, x_ref.shape[0]).T
    out_ref[...] = (contrib + b2_ref[0]) * gcol


def _ffn(disp, W1, b1, W2, b2, gfs, C, eo_rows):
  E, M, F = W1.shape
  ecl = lambda e: jnp.minimum(e, E - 1)
  return pl.pallas_call(
      functools.partial(_ffn_body, E),
      grid=(E + 1,),
      in_specs=[
          pl.BlockSpec((C, M), lambda e: (ecl(e), 0)),
          pl.BlockSpec((1, M, F), lambda e: (ecl(e), 0, 0)),
          pl.BlockSpec((1, 1, F), lambda e: (ecl(e), 0, 0)),
          pl.BlockSpec((1, F, M), lambda e: (ecl(e), 0, 0)),
          pl.BlockSpec((1, 1, M), lambda e: (ecl(e), 0, 0)),
          pl.BlockSpec((C,), lambda e: (ecl(e),)),
      ],
      out_specs=pl.BlockSpec((C, M), lambda e: (e, 0)),
      out_shape=jax.ShapeDtypeStruct((eo_rows, M), jnp.float32),
  )(disp, W1, b1.reshape(E, 1, F), W2, b2.reshape(E, 1, M), gfs)


_NC = 2
_NS = 16
_NW = _NC * _NS


def _make_dispatch(S, M, n_table, slot_base, n_half, CH):
  mesh = plsc.VectorSubcoreMesh(core_axis_name="c", subcore_axis_name="s")
  TPT = S // _NS
  SPW = n_half // _NW
  TFSN = ((n_table + _NS * 16 - 1) // (_NS * 16)) * _NS * 16
  ZPT = TFSN // _NS

  @functools.partial(
      pl.kernel,
      mesh=mesh,
      out_type=(
          jax.ShapeDtypeStruct((n_half, M), jnp.float32),
          jax.ShapeDtypeStruct((n_half,), jnp.float32),
      ),
      scratch_types=[
          pltpu.VMEM_SHARED((TFSN,), jnp.int32),
          pltpu.VMEM((ZPT,), jnp.int32),
          pltpu.VMEM((2, TPT // 2), jnp.int32),
          pltpu.VMEM((2, TPT // 2), jnp.int32),
          pltpu.VMEM((SPW,), jnp.int32),
          pltpu.VMEM((SPW,), jnp.int32),
          pltpu.VMEM((SPW,), jnp.float32),
          pltpu.VMEM((CH, M), jnp.float32),
          pltpu.VMEM((CH, M), jnp.float32),
          pltpu.SemaphoreType.DMA,
          pltpu.SemaphoreType.DMA,
          pltpu.SemaphoreType.DMA,
          pltpu.SemaphoreType.DMA,
          pltpu.SemaphoreType.DMA,
      ],
  )
  def dispatch(x_hbm, slot_hbm, scale_hbm, out_hbm, gfs_hbm, tfs_sh, zb_v,
               slots_v, ids_v, t_v, idxg_v, gsc_v, r0, r1, l0, l1, s0, s1,
               gs):
    cid = lax.axis_index("c")
    sid = lax.axis_index("s")
    wid = sid * _NC + cid
    for i in range(ZPT // 16):
      zb_v[pl.ds(i * 16, 16)] = jnp.zeros((16,), jnp.int32)
    pltpu.sync_copy(zb_v, tfs_sh.at[pl.ds(sid * ZPT, ZPT)])
    for k in range(2):
      pltpu.sync_copy(
          slot_hbm.at[pl.ds(sid * TPT + k * (TPT // 2), TPT // 2)],
          slots_v.at[k])
      for v in range(TPT // 32):
        ids_v[k, pl.ds(v * 16, 16)] = (
            lax.iota(jnp.int32, 16) + (sid * TPT + k * (TPT // 2) + v * 16
                                       + 1))
    plsc.subcore_barrier()
    for k in range(2):
      pltpu.sync_copy(ids_v.at[k], tfs_sh.at[slots_v.at[k]], add=True)
    plsc.subcore_barrier()
    sbase = wid * SPW
    pltpu.sync_copy(tfs_sh.at[pl.ds(slot_base + sbase, SPW)], t_v)
    for v in range(SPW // 16):
      tv = t_v[pl.ds(v * 16, 16)]
      idxg_v[pl.ds(v * 16, 16)] = jnp.maximum(tv, 1) - 1
    gcopy = pltpu.async_copy(scale_hbm.at[idxg_v], gsc_v, gs)
    bufs, lsem, ssem = (r0, r1), (l0, l1), (s0, s1)
    stores = [None, None]
    for j in range(SPW // CH):
      b = j % 2
      if stores[b] is not None:
        stores[b].wait()
      pltpu.async_copy(x_hbm.at[idxg_v.at[pl.ds(j * CH, CH)]], bufs[b],
                       lsem[b]).wait()
      stores[b] = pltpu.async_copy(bufs[b],
                                   out_hbm.at[pl.ds(sbase + j * CH, CH)],
                                   ssem[b])
    gcopy.wait()
    pltpu.sync_copy(gsc_v, gfs_hbm.at[pl.ds(sbase, SPW)])
    for s in stores:
      s.wait()

  return dispatch


def _make_decode(S, M, K, CH):
  mesh = plsc.VectorSubcoreMesh(core_axis_name="c", subcore_axis_name="s")

  @functools.partial(
      pl.kernel,
      mesh=mesh,
      out_type=jax.ShapeDtypeStruct((S, M), jnp.float32),
      scratch_types=[
          pltpu.VMEM((K * CH,), jnp.int32),
          pltpu.VMEM((CH, M), jnp.float32),
          pltpu.VMEM((CH, M), jnp.float32),
          pltpu.SemaphoreType.DMA,
          pltpu.SemaphoreType.DMA,
          pltpu.SemaphoreType.DMA,
          pltpu.SemaphoreType.DMA,
      ],
  )
  def decode(eo_hbm, slot_hbm, out_hbm, idx_v, r0, r1, g0, g1, s0, s1):
    wid = lax.axis_index("s") * _NC + lax.axis_index("c")
    tbase = wid * (K * CH)
    pltpu.sync_copy(slot_hbm.at[pl.ds(tbase, K * CH)], idx_v)
    bufs, gsem, ssem = (r0, r1), (g0, g1), (s0, s1)
    stores = [None, None]
    for j in range(K):
      b = j % 2
      if stores[b] is not None:
        stores[b].wait()
      base = wid * (K * CH) + j * CH
      pltpu.async_copy(eo_hbm.at[idx_v.at[pl.ds(j * CH, CH)]], bufs[b],
                       gsem[b]).wait()
      stores[b] = pltpu.async_copy(bufs[b], out_hbm.at[pl.ds(base, CH)],
                                   ssem[b])
    for s in stores:
      s.wait()

  return decode


def kernel(x, Wg, bg, W1, b1, W2, b2):
  orig_shape = x.shape
  M = x.shape[-1]
  xr = x.reshape(-1, M)
  S = xr.shape[0]
  E = Wg.shape[1]
  C = (S + E - 1) // E
  n_table = E * C + 8
  eo_rows = E * C + 8

  K, CH = 4, 32
  assert S == _NW * K * CH

  slot, scale = _route(xr, Wg, bg, C, E, block_t=1024)
  slot1 = slot.reshape(S)
  scale1 = scale.reshape(S)

  dispF, gfsF = _make_dispatch(S, M, n_table, 0, S, CH)(xr, slot1, scale1)
  eo = _ffn(dispF, W1, b1, W2, b2, gfsF, C, eo_rows)
  rout = _make_decode(S, M, K, CH)(eo, slot1)
  return rout.reshape(orig_shape)
